# baseline (speedup 1.0000x reference)
import math

import jax
import jax.numpy as jnp
from jax import lax
from jax.experimental import pallas as pl
from jax.experimental.pallas import tpu as pltpu
from jax.experimental.pallas import tpu_sc as plsc

DIM = 64
SCALE = math.sqrt(DIM)
LANES = 16
CHUNK = 128


def _make_kernel(num_workers: int, steps: int):
    total_rows = num_workers * steps * CHUNK
    mesh = plsc.VectorSubcoreMesh(core_axis_name="c", subcore_axis_name="s")

    def body(idx_hbm, table_hbm, out_hbm,
             idx_v, g0, g1, o0, o1, sg0, sg1, so0, so1):
        nc = mesh.num_cores
        wid = lax.axis_index("s") * nc + lax.axis_index("c")
        base = wid * (steps * CHUNK)
        gb = (g0, g1)
        ob = (o0, o1)
        sg = (sg0, sg1)
        so = (so0, so1)

        pltpu.sync_copy(idx_hbm.at[wid], idx_v)

        def start_gather(j, b):
            pltpu.async_copy(table_hbm.at[idx_v.at[j]], gb[b], sg[b])

        def scale(b):
            @pl.loop(0, CHUNK, unroll=4)
            def _(r):
                for k in range(DIM // LANES):
                    sl = pl.ds(k * LANES, LANES)
                    ob[b][r, sl] = gb[b][r, sl] * SCALE

        def pipe_step(j, b, *, out_wait, prefetch):
            pltpu.make_async_copy(table_hbm.at[idx_v.at[j]], gb[b], sg[b]).wait()
            if out_wait:
                pltpu.make_async_copy(
                    ob[b], out_hbm.at[pl.ds(base, CHUNK)], so[b]).wait()
            scale(b)
            if prefetch:
                start_gather(j + 2, b)
            pltpu.async_copy(ob[b], out_hbm.at[pl.ds(base + j * CHUNK, CHUNK)],
                             so[b])

        start_gather(0, 0)
        start_gather(1, 1)
        pipe_step(0, 0, out_wait=False, prefetch=True)
        pipe_step(1, 1, out_wait=False, prefetch=True)

        @pl.loop(1, steps // 2 - 1)
        def _(g):
            pipe_step(2 * g, 0, out_wait=True, prefetch=True)
            pipe_step(2 * g + 1, 1, out_wait=True, prefetch=True)

        pipe_step(steps - 2, 0, out_wait=True, prefetch=False)
        pipe_step(steps - 1, 1, out_wait=True, prefetch=False)
        pltpu.make_async_copy(o0, out_hbm.at[pl.ds(base, CHUNK)], so0).wait()
        pltpu.make_async_copy(o1, out_hbm.at[pl.ds(base, CHUNK)], so1).wait()

    kern = pl.kernel(
        body,
        out_type=jax.ShapeDtypeStruct((total_rows, DIM), jnp.float32),
        mesh=mesh,
        compiler_params=pltpu.CompilerParams(use_tc_tiling_on_sc=False),
        scratch_types=[
            pltpu.VMEM((steps, CHUNK), jnp.int32),
            pltpu.VMEM((CHUNK, DIM), jnp.float32),
            pltpu.VMEM((CHUNK, DIM), jnp.float32),
            pltpu.VMEM((CHUNK, DIM), jnp.float32),
            pltpu.VMEM((CHUNK, DIM), jnp.float32),
            pltpu.SemaphoreType.DMA,
            pltpu.SemaphoreType.DMA,
            pltpu.SemaphoreType.DMA,
            pltpu.SemaphoreType.DMA,
        ],
    )
    return kern


def kernel(x, w_embed):
    batch, hist = x.shape
    total = batch * hist
    info = plsc.get_sparse_core_info()
    num_workers = info.num_cores * info.num_subcores
    steps = total // (num_workers * CHUNK)
    assert steps * num_workers * CHUNK == total
    idx = x.reshape(num_workers, steps, CHUNK).astype(jnp.int32)
    out = _make_kernel(num_workers, steps)(idx, w_embed)
    return out.reshape(batch, hist, DIM)
